# baseline (speedup 1.0000x reference)
import functools

import jax
import jax.numpy as jnp
from jax import lax
from jax.experimental import pallas as pl
from jax.experimental.pallas import tpu as pltpu
import jax.experimental.pallas.tpu_sc as plsc

N = 10000
E = 160000
H = 128
NB = 2
NC = 4

KC = 80
CHC = 64
KP = 128
CHP = 40
E_PAD = 2 * 16 * CHC * KC
PAD = E_PAD - E
AGG_ROWS = 10240
NEG = -1e9

EBLK = 8192
BN = 2000


def _bf16_dot(a, b):
    return jnp.dot(a.astype(jnp.bfloat16), b.astype(jnp.bfloat16),
                   preferred_element_type=jnp.float32)


_SC_MESH = dict(core_axis_name="c", subcore_axis_name="s")
_SC_PARAMS = pltpu.CompilerParams(use_tc_tiling_on_sc=False)


def _edge_encoder_body(d_ref, et_ref, elW1_ref, elb1_ref, elW2_ref, elb2_ref,
                       etemb_ref, W1c_ref, gmb1_ref, attr_ref, C_ref, el_ref):
    d = d_ref[...]
    el = jnp.sqrt(jnp.sum(d * d, axis=1, keepdims=True) + 1e-12)
    eh = jnp.maximum(el * elW1_ref[...] + elb1_ref[...], 0.0)
    eh = _bf16_dot(eh, elW2_ref[...]) + elb2_ref[...]
    et = et_ref[0, 0]
    onehot = (et[:, None] == lax.broadcasted_iota(jnp.int32, (1, 8), 1)
              ).astype(jnp.float32)
    emb = lax.dot(onehot, etemb_ref[...],
                  precision=lax.Precision.HIGHEST)
    attr = eh * emb
    gid = pl.program_id(0) * EBLK + lax.broadcasted_iota(jnp.int32, (EBLK, 1), 0)
    attr = jnp.where(gid < E, attr, NEG)
    attr_ref[...] = attr
    C_ref[...] = _bf16_dot(attr, W1c_ref[...]) + gmb1_ref[...]
    el_ref[...] = el


def _edge_encoder(d016, et_pad, el_W1, el_b1, el_W2, el_b2, et_emb, W1c, gm_b1):
    nblk = E_PAD // EBLK
    et3 = et_pad.reshape(nblk, 1, EBLK)
    return pl.pallas_call(
        _edge_encoder_body,
        grid=(nblk,),
        in_specs=[
            pl.BlockSpec((EBLK, 16), lambda i: (i, 0)),
            pl.BlockSpec((1, 1, EBLK), lambda i: (i, 0, 0)),
            pl.BlockSpec((1, H), lambda i: (0, 0)),
            pl.BlockSpec((H,), lambda i: (0,)),
            pl.BlockSpec((H, H), lambda i: (0, 0)),
            pl.BlockSpec((H,), lambda i: (0,)),
            pl.BlockSpec((8, H), lambda i: (0, 0)),
            pl.BlockSpec((H, H), lambda i: (0, 0)),
            pl.BlockSpec((H,), lambda i: (0,)),
        ],
        out_specs=[
            pl.BlockSpec((EBLK, H), lambda i: (i, 0)),
            pl.BlockSpec((EBLK, H), lambda i: (i, 0)),
            pl.BlockSpec((EBLK, 1), lambda i: (i, 0)),
        ],
        out_shape=[
            jax.ShapeDtypeStruct((E_PAD, H), jnp.float32),
            jax.ShapeDtypeStruct((E_PAD, H), jnp.float32),
            jax.ShapeDtypeStruct((E_PAD, 1), jnp.float32),
        ],
    )(d016, et3, el_W1, el_b1, el_W2, el_b2, et_emb, W1c, gm_b1)


def _deg_embed_body(nd_ref, emb_ref, z_ref):
    nd = nd_ref[0, 0]
    onehot = (nd[:, None] == lax.broadcasted_iota(jnp.int32, (1, 64), 1)
              ).astype(jnp.float32)
    z_ref[...] = lax.dot(onehot, emb_ref[...], precision=lax.Precision.HIGHEST)


def _deg_embed(node_degree, deg_emb):
    nd3 = node_degree.astype(jnp.int32).reshape(N // BN, 1, BN)
    return pl.pallas_call(
        _deg_embed_body,
        grid=(N // BN,),
        in_specs=[
            pl.BlockSpec((1, 1, BN), lambda i: (i, 0, 0)),
            pl.BlockSpec((64, H), lambda i: (0, 0)),
        ],
        out_specs=pl.BlockSpec((BN, H), lambda i: (i, 0)),
        out_shape=jax.ShapeDtypeStruct((N, H), jnp.float32),
    )(nd3, deg_emb)


def _sc_conv_agg(h, attr_pad, row3c, col3c):
    @functools.partial(
        pl.kernel,
        out_type=jax.ShapeDtypeStruct((2, AGG_ROWS, H), jnp.float32),
        mesh=plsc.VectorSubcoreMesh(**_SC_MESH),
        scratch_types=[
            pltpu.VMEM((4, KC), jnp.int32),
            pltpu.VMEM((4, KC), jnp.int32),
            pltpu.VMEM((4, KC, H), jnp.float32),
            pltpu.VMEM_SHARED((AGG_ROWS, H), jnp.float32),
            pltpu.SemaphoreType.DMA,
            pltpu.SemaphoreType.DMA,
            pltpu.SemaphoreType.DMA,
            pltpu.SemaphoreType.DMA,
            pltpu.SemaphoreType.DMA,
            pltpu.SemaphoreType.DMA,
            pltpu.SemaphoreType.DMA,
            pltpu.SemaphoreType.DMA,
            pltpu.SemaphoreType.DMA,
            pltpu.SemaphoreType.DMA,
            pltpu.SemaphoreType.DMA,
            pltpu.SemaphoreType.DMA,
        ],
    )
    def k(h_hbm, attr_hbm, row_hbm, col_hbm, out_hbm,
          idx_r, idx_c, rows_v, agg_s, g0, g1, g2, g3, a0, a1, a2, a3,
          s0, s1, s2, s3_):
        cid = lax.axis_index("c")
        sid = lax.axis_index("s")
        wbase = (cid * 16 + sid) * CHC
        sem_g = (g0, g1, g2, g3)
        sem_a = (a0, a1, a2, a3)
        sem_s = (s0, s1, s2, s3_)
        zero = jnp.zeros((16,), jnp.float32)

        def zbody(i, carry):
            for l in range(8):
                rows_v[0, i, pl.ds(l * 16, 16)] = zero
            return carry
        lax.fori_loop(0, KC, zbody, 0)
        for kk in range(640 // KC):
            pltpu.sync_copy(rows_v.at[0], agg_s.at[pl.ds(sid * 640 + kk * KC, KC)])
        plsc.subcore_barrier()

        def load_attr_idx(j, s):
            pltpu.async_copy(attr_hbm.at[pl.ds((wbase + j) * KC, KC)],
                             rows_v.at[s], sem_a[s])
            pltpu.async_copy(row_hbm.at[cid, sid, j], idx_r.at[s], sem_a[s])
            pltpu.async_copy(col_hbm.at[cid, sid, j], idx_c.at[s], sem_a[s])

        def start_gather(j, s):
            pltpu.make_async_copy(attr_hbm.at[pl.ds((wbase + j) * KC, KC)],
                                  rows_v.at[s], sem_a[s]).wait()
            pltpu.make_async_copy(row_hbm.at[cid, sid, j],
                                  idx_r.at[s], sem_a[s]).wait()
            pltpu.make_async_copy(col_hbm.at[cid, sid, j],
                                  idx_c.at[s], sem_a[s]).wait()
            pltpu.async_copy(h_hbm.at[idx_r.at[s]], rows_v.at[s],
                             sem_g[s], add=True)

        for jj in range(3):
            load_attr_idx(jj, jj)
        for jj in range(2):
            start_gather(jj, jj)

        def body(j4, carry):
            for s in range(4):
                j = j4 * 4 + s
                pltpu.make_async_copy(h_hbm.at[idx_r.at[s]],
                                      rows_v.at[s], sem_g[s]).wait()

                def mbody(i, c2):
                    for l in range(8):
                        sl = pl.ds(l * 16, 16)
                        rows_v[s, i, sl] = jnp.maximum(rows_v[s, i, sl], 0.0)
                    return c2
                lax.fori_loop(0, KC, mbody, 0)
                pltpu.async_copy(rows_v.at[s], agg_s.at[idx_c.at[s]],
                                 sem_s[s], add=True)

                s3 = (s + 3) % 4

                @pl.when(j + 3 < CHC)
                def _():
                    @pl.when(j >= 1)
                    def _():
                        pltpu.make_async_copy(rows_v.at[s3],
                                              agg_s.at[idx_c.at[s3]],
                                              sem_s[s3]).wait()
                    load_attr_idx(j + 3, s3)

                @pl.when(j + 2 < CHC)
                def _():
                    start_gather(j + 2, (s + 2) % 4)
            return carry
        lax.fori_loop(0, CHC // 4, body, 0)
        for s in range(4):
            pltpu.make_async_copy(rows_v.at[s], agg_s.at[idx_c.at[s]],
                                  sem_s[s]).wait()
        plsc.subcore_barrier()
        pltpu.sync_copy(agg_s.at[pl.ds(sid * 640, 640)],
                        out_hbm.at[cid, pl.ds(sid * 640, 640)])

    return k(h, attr_pad, row3c, col3c)


def _sc_pos_diff(pf16, packed3):
    @functools.partial(
        pl.kernel,
        out_type=jax.ShapeDtypeStruct((E_PAD, 16), jnp.float32),
        mesh=plsc.VectorSubcoreMesh(**_SC_MESH),
        compiler_params=_SC_PARAMS,
        scratch_types=[
            pltpu.VMEM((CHP, KP), jnp.int32),
            pltpu.VMEM((2, KP), jnp.int32),
            pltpu.VMEM((2, KP), jnp.int32),
            pltpu.VMEM((2, KP, 16), jnp.float32),
            pltpu.VMEM((2, KP, 16), jnp.float32),
            pltpu.SemaphoreType.DMA,
            pltpu.SemaphoreType.DMA,
            pltpu.SemaphoreType.DMA,
            pltpu.SemaphoreType.DMA,
        ],
    )
    def k(pf_hbm, packed_hbm, out_hbm,
          packed_v, idx_r, idx_c, a_v, b_v, g0, g1, h0, h1):
        cid = lax.axis_index("c")
        sid = lax.axis_index("s")
        wbase = (cid * 16 + sid) * CHP
        sem_a = (g0, g1)
        sem_b = (h0, h1)
        pltpu.sync_copy(packed_hbm.at[cid, sid], packed_v)

        def start(j, b):
            for l in range(KP // 16):
                s = pl.ds(l * 16, 16)
                p = packed_v[j, s]
                idx_r[b, s] = jnp.bitwise_and(p, 0xFFFF)
                idx_c[b, s] = lax.shift_right_logical(p, 16)
            pltpu.async_copy(pf_hbm.at[idx_r.at[b]], a_v.at[b], sem_a[b])
            pltpu.async_copy(pf_hbm.at[idx_c.at[b]], b_v.at[b], sem_b[b])

        start(0, 0)
        start(1, 1)

        def body(j2, carry):
            for b in range(2):
                j = j2 * 2 + b
                pltpu.make_async_copy(pf_hbm.at[idx_r.at[b]],
                                      a_v.at[b], sem_a[b]).wait()
                pltpu.make_async_copy(pf_hbm.at[idx_c.at[b]],
                                      b_v.at[b], sem_b[b]).wait()

                def mbody(i, c2):
                    s = pl.ds(0, 16)
                    a_v[b, i, s] = a_v[b, i, s] - b_v[b, i, s]
                    return c2
                lax.fori_loop(0, KP, mbody, 0)
                pltpu.sync_copy(a_v.at[b], out_hbm.at[pl.ds((wbase + j) * KP, KP)])

                @pl.when(j + 2 < CHP)
                def _():
                    start(j + 2, b)
            return carry
        lax.fori_loop(0, CHP // 2, body, 0)

    return k(pf16, packed3)


def _sc_pair(A, Bm, C, packed3):
    @functools.partial(
        pl.kernel,
        out_type=jax.ShapeDtypeStruct((E_PAD, H), jnp.float32),
        mesh=plsc.VectorSubcoreMesh(**_SC_MESH),
        scratch_types=[
            pltpu.VMEM((CHP, KP), jnp.int32),
            pltpu.VMEM((2, KP), jnp.int32),
            pltpu.VMEM((2, KP), jnp.int32),
            pltpu.VMEM((2, KP, H), jnp.float32),
            pltpu.VMEM((2, KP, H), jnp.float32),
            pltpu.SemaphoreType.DMA,
            pltpu.SemaphoreType.DMA,
            pltpu.SemaphoreType.DMA,
            pltpu.SemaphoreType.DMA,
        ],
    )
    def k(A_hbm, B_hbm, C_hbm, packed_hbm, out_hbm,
          packed_v, idx_r, idx_c, a_v, b_v, s0, s1, s2, s3):
        cid = lax.axis_index("c")
        sid = lax.axis_index("s")
        wbase = (cid * 16 + sid) * CHP
        sem_a = (s0, s1)
        sem_b = (s2, s3)
        pltpu.sync_copy(packed_hbm.at[cid, sid], packed_v)

        def start(j, b):
            for l in range(KP // 16):
                s = pl.ds(l * 16, 16)
                p = packed_v[j, s]
                idx_r[b, s] = jnp.bitwise_and(p, 0xFFFF)
                idx_c[b, s] = lax.shift_right_logical(p, 16)
            pltpu.async_copy(C_hbm.at[pl.ds((wbase + j) * KP, KP)],
                             a_v.at[b], sem_a[b])

        def start2(j, b):
            pltpu.make_async_copy(C_hbm.at[pl.ds((wbase + j) * KP, KP)],
                                  a_v.at[b], sem_a[b]).wait()
            pltpu.async_copy(A_hbm.at[idx_r.at[b]], a_v.at[b], sem_b[b],
                             add=True)
            pltpu.async_copy(B_hbm.at[idx_c.at[b]], a_v.at[b], sem_b[b],
                             add=True)

        start(0, 0)
        start(1, 1)
        start2(0, 0)

        def body(j2, carry):
            for b in range(2):
                j = j2 * 2 + b

                @pl.when(j + 1 < CHP)
                def _():
                    start2(j + 1, 1 - b)

                pltpu.make_async_copy(A_hbm.at[idx_r.at[b]],
                                      a_v.at[b], sem_b[b]).wait()
                pltpu.make_async_copy(B_hbm.at[idx_c.at[b]],
                                      a_v.at[b], sem_b[b]).wait()

                def mbody(i, c2):
                    for l in range(8):
                        s = pl.ds(l * 16, 16)
                        a_v[b, i, s] = jnp.maximum(a_v[b, i, s], 0.0)
                    return c2
                lax.fori_loop(0, KP, mbody, 0)
                pltpu.sync_copy(a_v.at[b], out_hbm.at[pl.ds((wbase + j) * KP, KP)])

                @pl.when(j + 2 < CHP)
                def _():
                    start(j + 2, b)
            return carry
        lax.fori_loop(0, CHP // 2, body, 0)

    return k(A, Bm, C, packed3)


def _sc_eq(pf16, iol16, packed3):
    @functools.partial(
        pl.kernel,
        out_type=jax.ShapeDtypeStruct((2, AGG_ROWS, 16), jnp.float32),
        mesh=plsc.VectorSubcoreMesh(**_SC_MESH),
        compiler_params=_SC_PARAMS,
        scratch_types=[
            pltpu.VMEM((CHP, KP), jnp.int32),
            pltpu.VMEM((2, KP), jnp.int32),
            pltpu.VMEM((2, KP), jnp.int32),
            pltpu.VMEM((2, KP, 16), jnp.float32),
            pltpu.VMEM((2, KP, 16), jnp.float32),
            pltpu.VMEM((2, KP, 16), jnp.float32),
            pltpu.VMEM_SHARED((AGG_ROWS, 16), jnp.float32),
            pltpu.SemaphoreType.DMA,
            pltpu.SemaphoreType.DMA,
            pltpu.SemaphoreType.DMA,
            pltpu.SemaphoreType.DMA,
            pltpu.SemaphoreType.DMA,
            pltpu.SemaphoreType.DMA,
        ],
    )
    def k(pf_hbm, iol_hbm, packed_hbm, out_hbm,
          packed_v, idx_r, idx_c, a_v, b_v, i_v, acc_s, s0, s1, s2, s3, s4, s5):
        cid = lax.axis_index("c")
        sid = lax.axis_index("s")
        wbase = (cid * 16 + sid) * CHP
        sem_a = (s0, s1)
        sem_b = (s2, s3)
        sem_i = (s4, s5)
        zero = jnp.zeros((16,), jnp.float32)

        def zbody(i, carry):
            a_v[0, i, pl.ds(0, 16)] = zero
            return carry
        lax.fori_loop(0, KP, zbody, 0)
        for kk in range(640 // KP):
            pltpu.sync_copy(a_v.at[0], acc_s.at[pl.ds(sid * 640 + kk * KP, KP)])
        plsc.subcore_barrier()
        pltpu.sync_copy(packed_hbm.at[cid, sid], packed_v)

        def start(j, b):
            for l in range(KP // 16):
                s = pl.ds(l * 16, 16)
                p = packed_v[j, s]
                idx_r[b, s] = jnp.bitwise_and(p, 0xFFFF)
                idx_c[b, s] = lax.shift_right_logical(p, 16)
            pltpu.async_copy(pf_hbm.at[idx_r.at[b]], a_v.at[b], sem_a[b])
            pltpu.async_copy(pf_hbm.at[idx_c.at[b]], b_v.at[b], sem_b[b])
            pltpu.async_copy(iol_hbm.at[pl.ds((wbase + j) * KP, KP)],
                             i_v.at[b], sem_i[b])

        start(0, 0)
        start(1, 1)

        def body(j2, carry):
            for b in range(2):
                j = j2 * 2 + b
                pltpu.make_async_copy(pf_hbm.at[idx_r.at[b]],
                                      a_v.at[b], sem_a[b]).wait()
                pltpu.make_async_copy(pf_hbm.at[idx_c.at[b]],
                                      b_v.at[b], sem_b[b]).wait()
                pltpu.make_async_copy(iol_hbm.at[pl.ds((wbase + j) * KP, KP)],
                                      i_v.at[b], sem_i[b]).wait()

                def mbody(i, c2):
                    s = pl.ds(0, 16)
                    u = (a_v[b, i, s] - b_v[b, i, s]) * i_v[b, i, s]
                    a_v[b, i, s] = u
                    b_v[b, i, s] = -u
                    return c2
                lax.fori_loop(0, KP, mbody, 0)
                pltpu.sync_copy(a_v.at[b], acc_s.at[idx_r.at[b]], add=True)
                pltpu.sync_copy(b_v.at[b], acc_s.at[idx_c.at[b]], add=True)

                @pl.when(j + 2 < CHP)
                def _():
                    start(j + 2, b)
            return carry
        lax.fori_loop(0, CHP // 2, body, 0)
        plsc.subcore_barrier()
        pltpu.sync_copy(acc_s.at[pl.ds(sid * 640, 640)],
                        out_hbm.at[cid, pl.ds(sid * 640, 640)])

    return k(pf16, iol16, packed3)


def _conv_mlp_body(h_ref, agg_ref, W1_ref, b1_ref, W2_ref, b2_ref, out_ref):
    h = h_ref[...]
    h_in = h + (agg_ref[0] + agg_ref[1])
    t = jnp.maximum(_bf16_dot(h_in, W1_ref[...]) + b1_ref[...], 0.0)
    h_out = _bf16_dot(t, W2_ref[...]) + b2_ref[...]
    out_ref[...] = h + jnp.maximum(h_out, 0.0)


def _conv_mlp(h, agg, W1, b1, W2, b2):
    return pl.pallas_call(
        _conv_mlp_body,
        grid=(N // BN,),
        in_specs=[
            pl.BlockSpec((BN, H), lambda i: (i, 0)),
            pl.BlockSpec((2, BN, H), lambda i: (0, i, 0)),
            pl.BlockSpec((H, H), lambda i: (0, 0)),
            pl.BlockSpec((H,), lambda i: (0,)),
            pl.BlockSpec((H, H), lambda i: (0, 0)),
            pl.BlockSpec((H,), lambda i: (0,)),
        ],
        out_specs=pl.BlockSpec((BN, H), lambda i: (i, 0)),
        out_shape=jax.ShapeDtypeStruct((N, H), jnp.float32),
    )(h, agg, W1, b1, W2, b2)


def _conv_mlp_ab_body(h_ref, agg_ref, W1_ref, b1_ref, W2_ref, b2_ref,
                      W1a_ref, W1b_ref, out_ref, A_ref, B_ref):
    h = h_ref[...]
    h_in = h + (agg_ref[0] + agg_ref[1])
    t = jnp.maximum(_bf16_dot(h_in, W1_ref[...]) + b1_ref[...], 0.0)
    h_out = _bf16_dot(t, W2_ref[...]) + b2_ref[...]
    hn = h + jnp.maximum(h_out, 0.0)
    out_ref[...] = hn
    A_ref[...] = _bf16_dot(hn, W1a_ref[...])
    B_ref[...] = _bf16_dot(hn, W1b_ref[...])


def _conv_mlp_ab(h, agg, W1, b1, W2, b2, W1a, W1b):
    return pl.pallas_call(
        _conv_mlp_ab_body,
        grid=(N // BN,),
        in_specs=[
            pl.BlockSpec((BN, H), lambda i: (i, 0)),
            pl.BlockSpec((2, BN, H), lambda i: (0, i, 0)),
            pl.BlockSpec((H, H), lambda i: (0, 0)),
            pl.BlockSpec((H,), lambda i: (0,)),
            pl.BlockSpec((H, H), lambda i: (0, 0)),
            pl.BlockSpec((H,), lambda i: (0,)),
            pl.BlockSpec((H, H), lambda i: (0, 0)),
            pl.BlockSpec((H, H), lambda i: (0, 0)),
        ],
        out_specs=[
            pl.BlockSpec((BN, H), lambda i: (i, 0)),
            pl.BlockSpec((BN, H), lambda i: (i, 0)),
            pl.BlockSpec((BN, H), lambda i: (i, 0)),
        ],
        out_shape=[
            jax.ShapeDtypeStruct((N, H), jnp.float32),
            jax.ShapeDtypeStruct((N, H), jnp.float32),
            jax.ShapeDtypeStruct((N, H), jnp.float32),
        ],
    )(h, agg, W1, b1, W2, b2, W1a, W1b)


def _inv_body(e1_ref, el_ref, W2_ref, b2_ref, W3_ref, b3_ref, iol_ref):
    e2 = jnp.maximum(_bf16_dot(e1_ref[...], W2_ref[...]) + b2_ref[...], 0.0)
    inv = _bf16_dot(e2, W3_ref[...]) + b3_ref[...]
    iol = inv / el_ref[...]
    iol_ref[...] = jnp.broadcast_to(iol, (EBLK, 16))


def _inv_mlp(e1, el, gm_W2, gm_b2, gm_W3, gm_b3):
    return pl.pallas_call(
        _inv_body,
        grid=(E_PAD // EBLK,),
        in_specs=[
            pl.BlockSpec((EBLK, H), lambda i: (i, 0)),
            pl.BlockSpec((EBLK, 1), lambda i: (i, 0)),
            pl.BlockSpec((H, H // 2), lambda i: (0, 0)),
            pl.BlockSpec((H // 2,), lambda i: (0,)),
            pl.BlockSpec((H // 2, 1), lambda i: (0, 0)),
            pl.BlockSpec((1,), lambda i: (0,)),
        ],
        out_specs=pl.BlockSpec((EBLK, 16), lambda i: (i, 0)),
        out_shape=jax.ShapeDtypeStruct((E_PAD, 16), jnp.float32),
    )(e1, el, gm_W2, gm_b2, gm_W3, gm_b3)


def _pos_body(pf_ref, eq_ref, lm_ref, fm_ref, p0_ref, out_ref):
    pf = pf_ref[...] + (eq_ref[0] + eq_ref[1])
    out_ref[...] = pf * lm_ref[...] + p0_ref[...] * fm_ref[...]


def _pos_update(pf16, eqp, linker_mask, fragment_mask, pos016):
    return pl.pallas_call(
        _pos_body,
        grid=(N // BN,),
        in_specs=[
            pl.BlockSpec((BN, 16), lambda i: (i, 0)),
            pl.BlockSpec((2, BN, 16), lambda i: (0, i, 0)),
            pl.BlockSpec((BN, 1), lambda i: (i, 0)),
            pl.BlockSpec((BN, 1), lambda i: (i, 0)),
            pl.BlockSpec((BN, 16), lambda i: (i, 0)),
        ],
        out_specs=pl.BlockSpec((BN, 16), lambda i: (i, 0)),
        out_shape=jax.ShapeDtypeStruct((N, 16), jnp.float32),
    )(pf16, eqp, linker_mask, fragment_mask, pos016)


def kernel(node_emb, node_type, node_degree, pos, linker_mask, fragment_mask,
           edge_index, edge_type, batch, time_step,
           deg_emb, et_emb, el_W1, el_b1, el_W2, el_b2,
           gin_W1, gin_b1, gin_W2, gin_b2,
           gm_W1, gm_b1, gm_W2, gm_b2, gm_W3, gm_b3):
    row = edge_index[0].astype(jnp.int32)
    col = edge_index[1].astype(jnp.int32)
    packed = jnp.bitwise_or(row, jnp.left_shift(col, 16))
    packed_pad = jnp.concatenate([packed, jnp.zeros((PAD,), jnp.int32)])
    packed3p = packed_pad.reshape(2, 16, CHP, KP)
    row3c = jnp.concatenate([row, jnp.zeros((PAD,), jnp.int32)]
                            ).reshape(2, 16, CHC, KC)
    col3c = jnp.concatenate([col, jnp.zeros((PAD,), jnp.int32)]
                            ).reshape(2, 16, CHC, KC)
    et_pad = jnp.concatenate([edge_type.astype(jnp.int32),
                              jnp.zeros((PAD,), jnp.int32)])
    pos016 = jnp.pad(pos, ((0, 0), (0, 13)))

    d016 = _sc_pos_diff(pos016, packed3p)
    edge_attr, C, el = _edge_encoder(d016, et_pad, el_W1, el_b1, el_W2, el_b2,
                                     et_emb, gm_W1[2 * H:], gm_b1)
    z = _deg_embed(node_degree, deg_emb)

    pf16 = pos016
    for b in range(NB):
        h = z
        for c in range(NC):
            agg = _sc_conv_agg(h, edge_attr, row3c, col3c)
            if c < NC - 1:
                h = _conv_mlp(h, agg, gin_W1[b, c], gin_b1[b, c],
                              gin_W2[b, c], gin_b2[b, c])
            else:
                h, A, Bm = _conv_mlp_ab(h, agg, gin_W1[b, c], gin_b1[b, c],
                                        gin_W2[b, c], gin_b2[b, c],
                                        gm_W1[:H], gm_W1[H:2 * H])
        e1 = _sc_pair(A, Bm, C, packed3p)
        iol16 = _inv_mlp(e1, el, gm_W2, gm_b2, gm_W3, gm_b3)
        eqp = _sc_eq(pf16, iol16, packed3p)
        pf16 = _pos_update(pf16, eqp, linker_mask, fragment_mask, pos016)
    return (pf16 - pos016)[:, :3]


# speedup vs baseline: 1.0365x; 1.0365x over previous
import functools

import jax
import jax.numpy as jnp
from jax import lax
from jax.experimental import pallas as pl
from jax.experimental.pallas import tpu as pltpu
import jax.experimental.pallas.tpu_sc as plsc

N = 10000
E = 160000
H = 128
NB = 2
NC = 4

KC = 80
CHC = 64
KP = 128
CHP = 40
E_PAD = 2 * 16 * CHC * KC
PAD = E_PAD - E
AGG_ROWS = 10240
NEG = -1e9

EBLK = 8192
BN = 2000


def _bf16_dot(a, b):
    return jnp.dot(a.astype(jnp.bfloat16), b.astype(jnp.bfloat16),
                   preferred_element_type=jnp.float32)


_SC_MESH = dict(core_axis_name="c", subcore_axis_name="s")
_SC_PARAMS = pltpu.CompilerParams(use_tc_tiling_on_sc=False)


def _edge_encoder_body(d_ref, et_ref, elW1_ref, elb1_ref, elW2_ref, elb2_ref,
                       etemb_ref, W1c_ref, gmb1_ref, attr_ref, C_ref, el_ref):
    d = d_ref[...]
    el = jnp.sqrt(jnp.sum(d * d, axis=1, keepdims=True) + 1e-12)
    eh = jnp.maximum(el * elW1_ref[...] + elb1_ref[...], 0.0)
    eh = _bf16_dot(eh, elW2_ref[...]) + elb2_ref[...]
    et = et_ref[0, 0]
    onehot = (et[:, None] == lax.broadcasted_iota(jnp.int32, (1, 8), 1)
              ).astype(jnp.float32)
    emb = lax.dot(onehot, etemb_ref[...],
                  precision=lax.Precision.HIGHEST)
    attr = eh * emb
    gid = pl.program_id(0) * EBLK + lax.broadcasted_iota(jnp.int32, (EBLK, 1), 0)
    attr = jnp.where(gid < E, attr, NEG)
    attr_ref[...] = attr
    C_ref[...] = _bf16_dot(attr, W1c_ref[...]) + gmb1_ref[...]
    el_ref[...] = el


def _edge_encoder(d016, et_pad, el_W1, el_b1, el_W2, el_b2, et_emb, W1c, gm_b1):
    nblk = E_PAD // EBLK
    et3 = et_pad.reshape(nblk, 1, EBLK)
    return pl.pallas_call(
        _edge_encoder_body,
        grid=(nblk,),
        in_specs=[
            pl.BlockSpec((EBLK, 16), lambda i: (i, 0)),
            pl.BlockSpec((1, 1, EBLK), lambda i: (i, 0, 0)),
            pl.BlockSpec((1, H), lambda i: (0, 0)),
            pl.BlockSpec((H,), lambda i: (0,)),
            pl.BlockSpec((H, H), lambda i: (0, 0)),
            pl.BlockSpec((H,), lambda i: (0,)),
            pl.BlockSpec((8, H), lambda i: (0, 0)),
            pl.BlockSpec((H, H), lambda i: (0, 0)),
            pl.BlockSpec((H,), lambda i: (0,)),
        ],
        out_specs=[
            pl.BlockSpec((EBLK, H), lambda i: (i, 0)),
            pl.BlockSpec((EBLK, H), lambda i: (i, 0)),
            pl.BlockSpec((EBLK, 1), lambda i: (i, 0)),
        ],
        out_shape=[
            jax.ShapeDtypeStruct((E_PAD, H), jnp.float32),
            jax.ShapeDtypeStruct((E_PAD, H), jnp.float32),
            jax.ShapeDtypeStruct((E_PAD, 1), jnp.float32),
        ],
    )(d016, et3, el_W1, el_b1, el_W2, el_b2, et_emb, W1c, gm_b1)


def _deg_embed_body(nd_ref, emb_ref, z_ref):
    nd = nd_ref[0, 0]
    onehot = (nd[:, None] == lax.broadcasted_iota(jnp.int32, (1, 64), 1)
              ).astype(jnp.float32)
    z_ref[...] = lax.dot(onehot, emb_ref[...], precision=lax.Precision.HIGHEST)


def _deg_embed(node_degree, deg_emb):
    nd3 = node_degree.astype(jnp.int32).reshape(N // BN, 1, BN)
    return pl.pallas_call(
        _deg_embed_body,
        grid=(N // BN,),
        in_specs=[
            pl.BlockSpec((1, 1, BN), lambda i: (i, 0, 0)),
            pl.BlockSpec((64, H), lambda i: (0, 0)),
        ],
        out_specs=pl.BlockSpec((BN, H), lambda i: (i, 0)),
        out_shape=jax.ShapeDtypeStruct((N, H), jnp.float32),
    )(nd3, deg_emb)


def _sc_conv_agg(h, attr_pad, row3c, col3c):
    @functools.partial(
        pl.kernel,
        out_type=jax.ShapeDtypeStruct((2, AGG_ROWS, H), jnp.float32),
        mesh=plsc.VectorSubcoreMesh(**_SC_MESH),
        scratch_types=[
            pltpu.VMEM((4, KC), jnp.int32),
            pltpu.VMEM((4, KC), jnp.int32),
            pltpu.VMEM((4, KC, H), jnp.float32),
            pltpu.VMEM_SHARED((AGG_ROWS, H), jnp.float32),
            pltpu.SemaphoreType.DMA,
            pltpu.SemaphoreType.DMA,
            pltpu.SemaphoreType.DMA,
            pltpu.SemaphoreType.DMA,
            pltpu.SemaphoreType.DMA,
            pltpu.SemaphoreType.DMA,
            pltpu.SemaphoreType.DMA,
            pltpu.SemaphoreType.DMA,
            pltpu.SemaphoreType.DMA,
            pltpu.SemaphoreType.DMA,
            pltpu.SemaphoreType.DMA,
            pltpu.SemaphoreType.DMA,
        ],
    )
    def k(h_hbm, attr_hbm, row_hbm, col_hbm, out_hbm,
          idx_r, idx_c, rows_v, agg_s, g0, g1, g2, g3, a0, a1, a2, a3,
          s0, s1, s2, s3_):
        cid = lax.axis_index("c")
        sid = lax.axis_index("s")
        wbase = (cid * 16 + sid) * CHC
        sem_g = (g0, g1, g2, g3)
        sem_a = (a0, a1, a2, a3)
        sem_s = (s0, s1, s2, s3_)
        zero = jnp.zeros((16,), jnp.float32)

        def zbody(i, carry):
            for l in range(8):
                rows_v[0, i, pl.ds(l * 16, 16)] = zero
            return carry
        lax.fori_loop(0, KC, zbody, 0)
        for kk in range(640 // KC):
            pltpu.sync_copy(rows_v.at[0], agg_s.at[pl.ds(sid * 640 + kk * KC, KC)])
        plsc.subcore_barrier()

        def load_attr_idx(j, s):
            pltpu.async_copy(attr_hbm.at[pl.ds((wbase + j) * KC, KC)],
                             rows_v.at[s], sem_a[s])
            pltpu.async_copy(row_hbm.at[cid, sid, j], idx_r.at[s], sem_a[s])
            pltpu.async_copy(col_hbm.at[cid, sid, j], idx_c.at[s], sem_a[s])

        def start_gather(j, s):
            pltpu.make_async_copy(attr_hbm.at[pl.ds((wbase + j) * KC, KC)],
                                  rows_v.at[s], sem_a[s]).wait()
            pltpu.make_async_copy(row_hbm.at[cid, sid, j],
                                  idx_r.at[s], sem_a[s]).wait()
            pltpu.make_async_copy(col_hbm.at[cid, sid, j],
                                  idx_c.at[s], sem_a[s]).wait()
            pltpu.async_copy(h_hbm.at[idx_r.at[s]], rows_v.at[s],
                             sem_g[s], add=True)

        for jj in range(3):
            load_attr_idx(jj, jj)
        for jj in range(2):
            start_gather(jj, jj)

        def body(j4, carry):
            for s in range(4):
                j = j4 * 4 + s
                pltpu.make_async_copy(h_hbm.at[idx_r.at[s]],
                                      rows_v.at[s], sem_g[s]).wait()

                def mbody(i, c2):
                    for l in range(8):
                        sl = pl.ds(l * 16, 16)
                        rows_v[s, i, sl] = jnp.maximum(rows_v[s, i, sl], 0.0)
                    return c2
                lax.fori_loop(0, KC, mbody, 0)
                pltpu.async_copy(rows_v.at[s], agg_s.at[idx_c.at[s]],
                                 sem_s[s], add=True)

                s3 = (s + 3) % 4

                @pl.when(j + 3 < CHC)
                def _():
                    @pl.when(j >= 1)
                    def _():
                        pltpu.make_async_copy(rows_v.at[s3],
                                              agg_s.at[idx_c.at[s3]],
                                              sem_s[s3]).wait()
                    load_attr_idx(j + 3, s3)

                @pl.when(j + 2 < CHC)
                def _():
                    start_gather(j + 2, (s + 2) % 4)
            return carry
        lax.fori_loop(0, CHC // 4, body, 0)
        for s in range(4):
            pltpu.make_async_copy(rows_v.at[s], agg_s.at[idx_c.at[s]],
                                  sem_s[s]).wait()
        plsc.subcore_barrier()
        pltpu.sync_copy(agg_s.at[pl.ds(sid * 640, 640)],
                        out_hbm.at[cid, pl.ds(sid * 640, 640)])

    return k(h, attr_pad, row3c, col3c)


def _sc_pos_diff(pf16, packed3):
    @functools.partial(
        pl.kernel,
        out_type=jax.ShapeDtypeStruct((E_PAD, 16), jnp.float32),
        mesh=plsc.VectorSubcoreMesh(**_SC_MESH),
        compiler_params=_SC_PARAMS,
        scratch_types=[
            pltpu.VMEM((CHP, KP), jnp.int32),
            pltpu.VMEM((2, KP), jnp.int32),
            pltpu.VMEM((2, KP), jnp.int32),
            pltpu.VMEM((2, KP, 16), jnp.float32),
            pltpu.VMEM((2, KP, 16), jnp.float32),
            pltpu.SemaphoreType.DMA,
            pltpu.SemaphoreType.DMA,
            pltpu.SemaphoreType.DMA,
            pltpu.SemaphoreType.DMA,
        ],
    )
    def k(pf_hbm, packed_hbm, out_hbm,
          packed_v, idx_r, idx_c, a_v, b_v, g0, g1, h0, h1):
        cid = lax.axis_index("c")
        sid = lax.axis_index("s")
        wbase = (cid * 16 + sid) * CHP
        sem_a = (g0, g1)
        sem_b = (h0, h1)
        pltpu.sync_copy(packed_hbm.at[cid, sid], packed_v)

        def start(j, b):
            for l in range(KP // 16):
                s = pl.ds(l * 16, 16)
                p = packed_v[j, s]
                idx_r[b, s] = jnp.bitwise_and(p, 0xFFFF)
                idx_c[b, s] = lax.shift_right_logical(p, 16)
            pltpu.async_copy(pf_hbm.at[idx_r.at[b]], a_v.at[b], sem_a[b])
            pltpu.async_copy(pf_hbm.at[idx_c.at[b]], b_v.at[b], sem_b[b])

        start(0, 0)
        start(1, 1)

        def body(j2, carry):
            for b in range(2):
                j = j2 * 2 + b
                pltpu.make_async_copy(pf_hbm.at[idx_r.at[b]],
                                      a_v.at[b], sem_a[b]).wait()
                pltpu.make_async_copy(pf_hbm.at[idx_c.at[b]],
                                      b_v.at[b], sem_b[b]).wait()

                def mbody(i, c2):
                    s = pl.ds(0, 16)
                    a_v[b, i, s] = a_v[b, i, s] - b_v[b, i, s]
                    return c2
                lax.fori_loop(0, KP, mbody, 0)
                pltpu.sync_copy(a_v.at[b], out_hbm.at[pl.ds((wbase + j) * KP, KP)])

                @pl.when(j + 2 < CHP)
                def _():
                    start(j + 2, b)
            return carry
        lax.fori_loop(0, CHP // 2, body, 0)

    return k(pf16, packed3)


def _sc_pair(A, Bm, packed3):
    @functools.partial(
        pl.kernel,
        out_type=jax.ShapeDtypeStruct((E_PAD, H), jnp.float32),
        mesh=plsc.VectorSubcoreMesh(**_SC_MESH),
        scratch_types=[
            pltpu.VMEM((CHP, KP), jnp.int32),
            pltpu.VMEM((2, KP), jnp.int32),
            pltpu.VMEM((2, KP), jnp.int32),
            pltpu.VMEM((2, KP, H), jnp.float32),
            pltpu.VMEM((2, KP, H), jnp.float32),
            pltpu.SemaphoreType.DMA,
            pltpu.SemaphoreType.DMA,
            pltpu.SemaphoreType.DMA,
            pltpu.SemaphoreType.DMA,
        ],
    )
    def k(A_hbm, B_hbm, packed_hbm, out_hbm,
          packed_v, idx_r, idx_c, a_v, b_v, s0, s1, s2, s3):
        cid = lax.axis_index("c")
        sid = lax.axis_index("s")
        wbase = (cid * 16 + sid) * CHP
        sem_a = (s0, s1)
        sem_b = (s2, s3)
        pltpu.sync_copy(packed_hbm.at[cid, sid], packed_v)

        def start(j, b):
            for l in range(KP // 16):
                s = pl.ds(l * 16, 16)
                p = packed_v[j, s]
                idx_r[b, s] = jnp.bitwise_and(p, 0xFFFF)
                idx_c[b, s] = lax.shift_right_logical(p, 16)
            pltpu.async_copy(A_hbm.at[idx_r.at[b]], a_v.at[b], sem_a[b])
            pltpu.async_copy(B_hbm.at[idx_c.at[b]], b_v.at[b], sem_b[b])

        start(0, 0)
        start(1, 1)

        def body(j2, carry):
            for b in range(2):
                j = j2 * 2 + b
                pltpu.make_async_copy(A_hbm.at[idx_r.at[b]],
                                      a_v.at[b], sem_a[b]).wait()
                pltpu.make_async_copy(B_hbm.at[idx_c.at[b]],
                                      b_v.at[b], sem_b[b]).wait()

                def mbody(i, c2):
                    for l in range(8):
                        s = pl.ds(l * 16, 16)
                        a_v[b, i, s] = a_v[b, i, s] + b_v[b, i, s]
                    return c2
                lax.fori_loop(0, KP, mbody, 0)
                pltpu.sync_copy(a_v.at[b], out_hbm.at[pl.ds((wbase + j) * KP, KP)])

                @pl.when(j + 2 < CHP)
                def _():
                    start(j + 2, b)
            return carry
        lax.fori_loop(0, CHP // 2, body, 0)

    return k(A, Bm, packed3)


def _sc_eq(pf16, iol16, packed3):
    @functools.partial(
        pl.kernel,
        out_type=jax.ShapeDtypeStruct((2, AGG_ROWS, 16), jnp.float32),
        mesh=plsc.VectorSubcoreMesh(**_SC_MESH),
        compiler_params=_SC_PARAMS,
        scratch_types=[
            pltpu.VMEM((CHP, KP), jnp.int32),
            pltpu.VMEM((2, KP), jnp.int32),
            pltpu.VMEM((2, KP), jnp.int32),
            pltpu.VMEM((2, KP, 16), jnp.float32),
            pltpu.VMEM((2, KP, 16), jnp.float32),
            pltpu.VMEM((2, KP, 16), jnp.float32),
            pltpu.VMEM_SHARED((AGG_ROWS, 16), jnp.float32),
            pltpu.SemaphoreType.DMA,
            pltpu.SemaphoreType.DMA,
            pltpu.SemaphoreType.DMA,
            pltpu.SemaphoreType.DMA,
            pltpu.SemaphoreType.DMA,
            pltpu.SemaphoreType.DMA,
        ],
    )
    def k(pf_hbm, iol_hbm, packed_hbm, out_hbm,
          packed_v, idx_r, idx_c, a_v, b_v, i_v, acc_s, s0, s1, s2, s3, s4, s5):
        cid = lax.axis_index("c")
        sid = lax.axis_index("s")
        wbase = (cid * 16 + sid) * CHP
        sem_a = (s0, s1)
        sem_b = (s2, s3)
        sem_i = (s4, s5)
        zero = jnp.zeros((16,), jnp.float32)

        def zbody(i, carry):
            a_v[0, i, pl.ds(0, 16)] = zero
            return carry
        lax.fori_loop(0, KP, zbody, 0)
        for kk in range(640 // KP):
            pltpu.sync_copy(a_v.at[0], acc_s.at[pl.ds(sid * 640 + kk * KP, KP)])
        plsc.subcore_barrier()
        pltpu.sync_copy(packed_hbm.at[cid, sid], packed_v)

        def start(j, b):
            for l in range(KP // 16):
                s = pl.ds(l * 16, 16)
                p = packed_v[j, s]
                idx_r[b, s] = jnp.bitwise_and(p, 0xFFFF)
                idx_c[b, s] = lax.shift_right_logical(p, 16)
            pltpu.async_copy(pf_hbm.at[idx_r.at[b]], a_v.at[b], sem_a[b])
            pltpu.async_copy(pf_hbm.at[idx_c.at[b]], b_v.at[b], sem_b[b])
            pltpu.async_copy(iol_hbm.at[pl.ds((wbase + j) * KP, KP)],
                             i_v.at[b], sem_i[b])

        start(0, 0)
        start(1, 1)

        def body(j2, carry):
            for b in range(2):
                j = j2 * 2 + b
                pltpu.make_async_copy(pf_hbm.at[idx_r.at[b]],
                                      a_v.at[b], sem_a[b]).wait()
                pltpu.make_async_copy(pf_hbm.at[idx_c.at[b]],
                                      b_v.at[b], sem_b[b]).wait()
                pltpu.make_async_copy(iol_hbm.at[pl.ds((wbase + j) * KP, KP)],
                                      i_v.at[b], sem_i[b]).wait()

                def mbody(i, c2):
                    s = pl.ds(0, 16)
                    u = (a_v[b, i, s] - b_v[b, i, s]) * i_v[b, i, s]
                    a_v[b, i, s] = u
                    b_v[b, i, s] = -u
                    return c2
                lax.fori_loop(0, KP, mbody, 0)
                pltpu.sync_copy(a_v.at[b], acc_s.at[idx_r.at[b]], add=True)
                pltpu.sync_copy(b_v.at[b], acc_s.at[idx_c.at[b]], add=True)

                @pl.when(j + 2 < CHP)
                def _():
                    start(j + 2, b)
            return carry
        lax.fori_loop(0, CHP // 2, body, 0)
        plsc.subcore_barrier()
        pltpu.sync_copy(acc_s.at[pl.ds(sid * 640, 640)],
                        out_hbm.at[cid, pl.ds(sid * 640, 640)])

    return k(pf16, iol16, packed3)


def _conv_mlp_body(h_ref, agg_ref, W1_ref, b1_ref, W2_ref, b2_ref, out_ref):
    h = h_ref[...]
    h_in = h + (agg_ref[0] + agg_ref[1])
    t = jnp.maximum(_bf16_dot(h_in, W1_ref[...]) + b1_ref[...], 0.0)
    h_out = _bf16_dot(t, W2_ref[...]) + b2_ref[...]
    out_ref[...] = h + jnp.maximum(h_out, 0.0)


def _conv_mlp(h, agg, W1, b1, W2, b2):
    return pl.pallas_call(
        _conv_mlp_body,
        grid=(N // BN,),
        in_specs=[
            pl.BlockSpec((BN, H), lambda i: (i, 0)),
            pl.BlockSpec((2, BN, H), lambda i: (0, i, 0)),
            pl.BlockSpec((H, H), lambda i: (0, 0)),
            pl.BlockSpec((H,), lambda i: (0,)),
            pl.BlockSpec((H, H), lambda i: (0, 0)),
            pl.BlockSpec((H,), lambda i: (0,)),
        ],
        out_specs=pl.BlockSpec((BN, H), lambda i: (i, 0)),
        out_shape=jax.ShapeDtypeStruct((N, H), jnp.float32),
    )(h, agg, W1, b1, W2, b2)


def _conv_mlp_ab_body(h_ref, agg_ref, W1_ref, b1_ref, W2_ref, b2_ref,
                      W1a_ref, W1b_ref, out_ref, A_ref, B_ref):
    h = h_ref[...]
    h_in = h + (agg_ref[0] + agg_ref[1])
    t = jnp.maximum(_bf16_dot(h_in, W1_ref[...]) + b1_ref[...], 0.0)
    h_out = _bf16_dot(t, W2_ref[...]) + b2_ref[...]
    hn = h + jnp.maximum(h_out, 0.0)
    out_ref[...] = hn
    A_ref[...] = _bf16_dot(hn, W1a_ref[...])
    B_ref[...] = _bf16_dot(hn, W1b_ref[...])


def _conv_mlp_ab(h, agg, W1, b1, W2, b2, W1a, W1b):
    return pl.pallas_call(
        _conv_mlp_ab_body,
        grid=(N // BN,),
        in_specs=[
            pl.BlockSpec((BN, H), lambda i: (i, 0)),
            pl.BlockSpec((2, BN, H), lambda i: (0, i, 0)),
            pl.BlockSpec((H, H), lambda i: (0, 0)),
            pl.BlockSpec((H,), lambda i: (0,)),
            pl.BlockSpec((H, H), lambda i: (0, 0)),
            pl.BlockSpec((H,), lambda i: (0,)),
            pl.BlockSpec((H, H), lambda i: (0, 0)),
            pl.BlockSpec((H, H), lambda i: (0, 0)),
        ],
        out_specs=[
            pl.BlockSpec((BN, H), lambda i: (i, 0)),
            pl.BlockSpec((BN, H), lambda i: (i, 0)),
            pl.BlockSpec((BN, H), lambda i: (i, 0)),
        ],
        out_shape=[
            jax.ShapeDtypeStruct((N, H), jnp.float32),
            jax.ShapeDtypeStruct((N, H), jnp.float32),
            jax.ShapeDtypeStruct((N, H), jnp.float32),
        ],
    )(h, agg, W1, b1, W2, b2, W1a, W1b)


def _inv_body(g_ref, C_ref, el_ref, W2_ref, b2_ref, W3_ref, b3_ref, iol_ref):
    e1 = jnp.maximum(g_ref[...] + C_ref[...], 0.0)
    e2 = jnp.maximum(_bf16_dot(e1, W2_ref[...]) + b2_ref[...], 0.0)
    inv = _bf16_dot(e2, W3_ref[...]) + b3_ref[...]
    iol = inv / el_ref[...]
    iol_ref[...] = jnp.broadcast_to(iol, (EBLK, 16))


def _inv_mlp(g, C, el, gm_W2, gm_b2, gm_W3, gm_b3):
    return pl.pallas_call(
        _inv_body,
        grid=(E_PAD // EBLK,),
        in_specs=[
            pl.BlockSpec((EBLK, H), lambda i: (i, 0)),
            pl.BlockSpec((EBLK, H), lambda i: (i, 0)),
            pl.BlockSpec((EBLK, 1), lambda i: (i, 0)),
            pl.BlockSpec((H, H // 2), lambda i: (0, 0)),
            pl.BlockSpec((H // 2,), lambda i: (0,)),
            pl.BlockSpec((H // 2, 1), lambda i: (0, 0)),
            pl.BlockSpec((1,), lambda i: (0,)),
        ],
        out_specs=pl.BlockSpec((EBLK, 16), lambda i: (i, 0)),
        out_shape=jax.ShapeDtypeStruct((E_PAD, 16), jnp.float32),
    )(g, C, el, gm_W2, gm_b2, gm_W3, gm_b3)


def _pos_body(pf_ref, eq_ref, lm_ref, fm_ref, p0_ref, out_ref):
    pf = pf_ref[...] + (eq_ref[0] + eq_ref[1])
    out_ref[...] = pf * lm_ref[...] + p0_ref[...] * fm_ref[...]


def _pos_update(pf16, eqp, linker_mask, fragment_mask, pos016):
    return pl.pallas_call(
        _pos_body,
        grid=(N // BN,),
        in_specs=[
            pl.BlockSpec((BN, 16), lambda i: (i, 0)),
            pl.BlockSpec((2, BN, 16), lambda i: (0, i, 0)),
            pl.BlockSpec((BN, 1), lambda i: (i, 0)),
            pl.BlockSpec((BN, 1), lambda i: (i, 0)),
            pl.BlockSpec((BN, 16), lambda i: (i, 0)),
        ],
        out_specs=pl.BlockSpec((BN, 16), lambda i: (i, 0)),
        out_shape=jax.ShapeDtypeStruct((N, 16), jnp.float32),
    )(pf16, eqp, linker_mask, fragment_mask, pos016)


def kernel(node_emb, node_type, node_degree, pos, linker_mask, fragment_mask,
           edge_index, edge_type, batch, time_step,
           deg_emb, et_emb, el_W1, el_b1, el_W2, el_b2,
           gin_W1, gin_b1, gin_W2, gin_b2,
           gm_W1, gm_b1, gm_W2, gm_b2, gm_W3, gm_b3):
    row = edge_index[0].astype(jnp.int32)
    col = edge_index[1].astype(jnp.int32)
    packed = jnp.bitwise_or(row, jnp.left_shift(col, 16))
    packed_pad = jnp.concatenate([packed, jnp.zeros((PAD,), jnp.int32)])
    packed3p = packed_pad.reshape(2, 16, CHP, KP)
    row3c = jnp.concatenate([row, jnp.zeros((PAD,), jnp.int32)]
                            ).reshape(2, 16, CHC, KC)
    col3c = jnp.concatenate([col, jnp.zeros((PAD,), jnp.int32)]
                            ).reshape(2, 16, CHC, KC)
    et_pad = jnp.concatenate([edge_type.astype(jnp.int32),
                              jnp.zeros((PAD,), jnp.int32)])
    pos016 = jnp.pad(pos, ((0, 0), (0, 13)))

    d016 = _sc_pos_diff(pos016, packed3p)
    edge_attr, C, el = _edge_encoder(d016, et_pad, el_W1, el_b1, el_W2, el_b2,
                                     et_emb, gm_W1[2 * H:], gm_b1)
    z = _deg_embed(node_degree, deg_emb)

    pf16 = pos016
    for b in range(NB):
        h = z
        for c in range(NC):
            agg = _sc_conv_agg(h, edge_attr, row3c, col3c)
            if c < NC - 1:
                h = _conv_mlp(h, agg, gin_W1[b, c], gin_b1[b, c],
                              gin_W2[b, c], gin_b2[b, c])
            else:
                h, A, Bm = _conv_mlp_ab(h, agg, gin_W1[b, c], gin_b1[b, c],
                                        gin_W2[b, c], gin_b2[b, c],
                                        gm_W1[:H], gm_W1[H:2 * H])
        g = _sc_pair(A, Bm, packed3p)
        iol16 = _inv_mlp(g, C, el, gm_W2, gm_b2, gm_W3, gm_b3)
        eqp = _sc_eq(pf16, iol16, packed3p)
        pf16 = _pos_update(pf16, eqp, linker_mask, fragment_mask, pos016)
    return (pf16 - pos016)[:, :3]
